# baseline (speedup 1.0000x reference)
import functools

import jax
import jax.numpy as jnp
from jax import lax
from jax.experimental import pallas as pl
from jax.experimental.pallas import tpu as pltpu
from jax.experimental.pallas import tpu_sc as plsc

_G = 64
_RB = 2000
_EB = 80
_NSUB = 16
_NCORE = 2
_LANES = 16


def _tc_call(body, grid, in_specs, out_shape, out_specs):
    return pl.pallas_call(
        body, grid=grid, in_specs=in_specs, out_shape=out_shape,
        out_specs=out_specs)


def _enc_nodes(node, W_n1, b_n1, W_n2, b_n2):
    N = node.shape[0]
    H = W_n1.shape[1]
    D = W_n2.shape[1]

    def body(n_ref, w1_ref, b1_ref, w2_ref, b2_ref, o_ref):
        t = jnp.maximum(n_ref[...] * w1_ref[...] + b1_ref[...], 0.0)
        o_ref[...] = jnp.dot(t, w2_ref[...],
                             preferred_element_type=jnp.float32) + b2_ref[...]

    return _tc_call(
        body, (N // _RB,),
        [pl.BlockSpec((_RB, 1), lambda r: (r, 0)),
         pl.BlockSpec((1, H), lambda r: (0, 0)),
         pl.BlockSpec((1, H), lambda r: (0, 0)),
         pl.BlockSpec((H, D), lambda r: (0, 0)),
         pl.BlockSpec((1, D), lambda r: (0, 0))],
        jax.ShapeDtypeStruct((N, D), jnp.float32),
        pl.BlockSpec((_RB, D), lambda r: (r, 0)),
    )(node, W_n1, b_n1.reshape(1, H), W_n2, b_n2.reshape(1, D))


def _enc_edge_pre(edge, W_e1, b_e1):
    E = edge.shape[0]
    H = W_e1.shape[1]

    def body(e_ref, w1_ref, b1_ref, o_ref):
        o_ref[...] = jnp.maximum(e_ref[...] * w1_ref[...] + b1_ref[...], 0.0)

    return _tc_call(
        body, (E // _RB,),
        [pl.BlockSpec((_RB, 1), lambda r: (r, 0)),
         pl.BlockSpec((1, H), lambda r: (0, 0)),
         pl.BlockSpec((1, H), lambda r: (0, 0))],
        jax.ShapeDtypeStruct((E, H), jnp.float32),
        pl.BlockSpec((_RB, H), lambda r: (r, 0)),
    )(edge, W_e1, b_e1.reshape(1, H))


def _fold_edge_weights(W_e2, b_e2, W_ef, b_msg1):
    L, D, _ = W_ef.shape
    H = W_e2.shape[0]

    def body(we2_ref, wef_ref, be2_ref, bm1_ref, m_ref, c0_ref):
        wef = wef_ref[0]
        m_ref[...] = jnp.dot(we2_ref[...], wef,
                             preferred_element_type=jnp.float32)[None]
        c0_ref[...] = (jnp.dot(be2_ref[...], wef,
                               preferred_element_type=jnp.float32)
                       + bm1_ref[0])[None]

    return pl.pallas_call(
        body, grid=(L,),
        in_specs=[pl.BlockSpec((H, D), lambda i: (0, 0)),
                  pl.BlockSpec((1, D, D), lambda i: (i, 0, 0)),
                  pl.BlockSpec((1, D), lambda i: (0, 0)),
                  pl.BlockSpec((1, 1, D), lambda i: (i, 0, 0))],
        out_shape=[jax.ShapeDtypeStruct((L, H, D), jnp.float32),
                   jax.ShapeDtypeStruct((L, 1, D), jnp.float32)],
        out_specs=[pl.BlockSpec((1, H, D), lambda i: (i, 0, 0)),
                   pl.BlockSpec((1, 1, D), lambda i: (i, 0, 0))],
    )(W_e2, W_ef, b_e2.reshape(1, D), b_msg1.reshape(L, 1, D))


def _cmat(A, M_i, c0_i):
    E, H = A.shape
    D = M_i.shape[1]
    HALF = D // 2

    def body(a_ref, m_ref, c0_ref, o_ref):
        o_ref[...] = jnp.dot(a_ref[...], m_ref[...],
                             preferred_element_type=jnp.float32) + c0_ref[...]

    return _tc_call(
        body, (E // _RB, 2),
        [pl.BlockSpec((_RB, H), lambda r, c: (r, 0)),
         pl.BlockSpec((H, HALF), lambda r, c: (0, c)),
         pl.BlockSpec((1, HALF), lambda r, c: (0, c))],
        jax.ShapeDtypeStruct((2 * E, HALF), jnp.float32),
        pl.BlockSpec((_RB, HALF), lambda r, c: (c * (E // _RB) + r, 0)),
    )(A, M_i, c0_i)


def _pmat(h, W_src, W_dst):
    N, D = h.shape
    HALF = D // 2

    def body(h_ref, ws_ref, wd_ref, os_ref, od_ref):
        hb = h_ref[...]
        os_ref[...] = jnp.dot(hb, ws_ref[...],
                              preferred_element_type=jnp.float32)
        od_ref[...] = jnp.dot(hb, wd_ref[...],
                              preferred_element_type=jnp.float32)

    nb = N // _RB
    return pl.pallas_call(
        body, grid=(nb, 2),
        in_specs=[pl.BlockSpec((_RB, D), lambda r, c: (r, 0)),
                  pl.BlockSpec((D, HALF), lambda r, c: (0, c)),
                  pl.BlockSpec((D, HALF), lambda r, c: (0, c))],
        out_shape=[jax.ShapeDtypeStruct((2 * N, HALF), jnp.float32),
                   jax.ShapeDtypeStruct((2 * N, HALF), jnp.float32)],
        out_specs=[pl.BlockSpec((_RB, HALF), lambda r, c: (c * nb + r, 0)),
                   pl.BlockSpec((_RB, HALF), lambda r, c: (c * nb + r, 0))],
    )(h, W_src, W_dst)


def _sc_mesh():
    return plsc.VectorSubcoreMesh(core_axis_name="c", subcore_axis_name="s")


def _edge_stage(psrc, pdst, cmat, from_idx, to_idx, N, NP, E, HALF):
    RPT = NP // _NSUB
    ZR = 128
    EPT = E // _NSUB

    NB = EPT // _EB
    QB = 16
    NQ = _EB // QB

    @functools.partial(
        pl.kernel, mesh=_sc_mesh(),
        out_type=jax.ShapeDtypeStruct((2 * NP, HALF), jnp.float32),
        scratch_types=[
            pltpu.VMEM((_EB,), jnp.int32),
            pltpu.VMEM((_EB,), jnp.int32),
            pltpu.VMEM((_EB,), jnp.int32),
            pltpu.VMEM((_EB, HALF), jnp.float32),
            pltpu.VMEM((_EB, HALF), jnp.float32),
            pltpu.VMEM((_EB,), jnp.int32),
            pltpu.VMEM((_EB,), jnp.int32),
            pltpu.VMEM((_EB,), jnp.int32),
            pltpu.VMEM((_EB, HALF), jnp.float32),
            pltpu.VMEM((_EB, HALF), jnp.float32),
            pltpu.VMEM((_EB,), jnp.int32),
            pltpu.VMEM((_EB,), jnp.int32),
            pltpu.VMEM((16, HALF), jnp.float32),
            pltpu.VMEM((16, HALF), jnp.float32),
            pltpu.SemaphoreType.DMA,
            pltpu.SemaphoreType.DMA,
            pltpu.SemaphoreType.DMA,
            pltpu.SemaphoreType.DMA,
            pltpu.SemaphoreType.DMA,
            pltpu.SemaphoreType.DMA,
            pltpu.SemaphoreType.DMA,
            pltpu.SemaphoreType.DMA,
            pltpu.SemaphoreType.DMA,
            pltpu.VMEM_SHARED((NP, HALF), jnp.float32),
        ],
    )
    def k(psrc_hbm, pdst_hbm, c_hbm, f_hbm, t_hbm, s_hbm,
          fidxA, tidxA, didxA, sbufA, dbufA,
          fidxB, tidxB, didxB, sbufB, dbufB,
          stidxA, stidxB, cbuf0, cbuf1,
          slA, sgsA, sgdA, slB, sgsB, sgdB, scc, sscA, sscB, s_sp):
        c = lax.axis_index("c")
        s = lax.axis_index("s")
        noff = c * N
        poff = c * NP
        eoff = c * E
        ebase = s * EPT

        setA = (fidxA, tidxA, didxA, sbufA, dbufA, slA, sgsA, sgdA,
                stidxA, sscA)
        setB = (fidxB, tidxB, didxB, sbufB, dbufB, slB, sgsB, sgdB,
                stidxB, sscB)

        def scat_wait(st):
            pltpu.make_async_copy(st[3], s_sp.at[st[8]], st[9]).wait()

        def lin_copies(st, blk):
            fidx, tidx = st[0], st[1]
            sl = st[5]
            base = ebase + blk * _EB
            return (pltpu.make_async_copy(f_hbm.at[pl.ds(base, _EB)], fidx,
                                          sl),
                    pltpu.make_async_copy(t_hbm.at[pl.ds(base, _EB)], tidx,
                                          sl))

        def lin_start(st, blk):
            for cp in lin_copies(st, blk):
                cp.start()

        def lin_wait(st, blk):
            for cp in lin_copies(st, blk):
                cp.wait()

        def gather_copies(st):
            fidx, didx, sbuf, dbuf, sgs, sgd = (st[0], st[2], st[3], st[4],
                                                st[6], st[7])
            return (pltpu.make_async_copy(psrc_hbm.at[fidx], sbuf, sgs),
                    pltpu.make_async_copy(pdst_hbm.at[didx], dbuf, sgd))

        def adjust_and_issue(st, wait_scat):
            if isinstance(wait_scat, bool):
                if wait_scat:
                    scat_wait(st)
            else:
                @pl.when(wait_scat)
                def _():
                    scat_wait(st)

            fidx, tidx, didx = st[0], st[1], st[2]
            for i in range(0, _EB, _LANES):
                sl = pl.ds(i, _LANES)
                fidx[sl] = fidx[sl] + noff
                didx[sl] = tidx[sl] + noff
            for cp in gather_copies(st):
                cp.start()

        def c_copy(blk, q, buf):
            return pltpu.make_async_copy(
                c_hbm.at[pl.ds(eoff + ebase + blk * _EB + q * QB, QB)],
                buf, scc)

        def compute_scatter(st, blk, nxt_blk, prefetch_next, phase):
            tidx, sbuf, dbuf, stidx, ssc = st[1], st[3], st[4], st[8], st[9]
            bufs = (cbuf0, cbuf1)
            for cp in gather_copies(st):
                cp.wait()
            for i in range(0, _EB, _LANES):
                sl = pl.ds(i, _LANES)
                stidx[sl] = tidx[sl]
            for q in range(NQ):
                cb = bufs[(phase + q) % 2]
                nb = bufs[(phase + q + 1) % 2]
                c_copy(blk, q, cb).wait()
                if q < NQ - 1:
                    c_copy(blk, q + 1, nb).start()
                elif isinstance(prefetch_next, bool):
                    if prefetch_next:
                        c_copy(nxt_blk, 0, nb).start()
                else:
                    @pl.when(prefetch_next)
                    def _():
                        c_copy(nxt_blk, 0, nb).start()

                @pl.loop(0, QB, step=4)
                def _(e0):
                    for u in range(4):
                        e = e0 + (q * QB + u)
                        ec = e0 + u
                        for j in range(0, HALF, _LANES):
                            sl = pl.ds(j, _LANES)
                            sbuf[e, sl] = jnp.maximum(
                                sbuf[e, sl] + dbuf[e, sl] + cb[ec, sl], 0.0)

            pltpu.async_copy(sbuf, s_sp.at[stidx], ssc, add=True)

        @pl.loop(0, _EB)
        def _(r):
            for j in range(0, HALF, _LANES):
                sbufA[r, pl.ds(j, _LANES)] = jnp.zeros((_LANES,), jnp.float32)

        @pl.loop(0, RPT, step=_EB)
        def _(r0):
            pltpu.sync_copy(sbufA, s_sp.at[pl.ds(s * RPT + r0, _EB)])

        plsc.subcore_barrier()

        lin_start(setA, 0)
        c_copy(0, 0, cbuf0).start()
        lin_wait(setA, 0)
        adjust_and_issue(setA, False)
        compute_scatter(setA, 0, 1, True, 0)
        lin_start(setA, 1)
        lin_wait(setA, 1)
        adjust_and_issue(setA, True)
        lin_start(setB, 2)

        @pl.loop(0, NB - 1, step=2)
        def _(j):
            lin_wait(setB, 2 + j)
            adjust_and_issue(setB, j > 0)
            compute_scatter(setA, 1 + j, 2 + j, True, 1)

            @pl.when(j < NB - 3)
            def _():
                lin_start(setA, 3 + j)
                lin_wait(setA, 3 + j)
                adjust_and_issue(setA, True)

            compute_scatter(setB, 2 + j, 3 + j, j < NB - 3, 0)

            @pl.when(j < NB - 4)
            def _():
                lin_start(setB, 4 + j)

        scat_wait(setA)
        scat_wait(setB)
        plsc.subcore_barrier()

        @pl.loop(0, RPT, step=ZR)
        def _(r0):
            r = s * RPT + r0
            pltpu.sync_copy(s_sp.at[pl.ds(r, ZR)],
                            s_hbm.at[pl.ds(poff + r, ZR)])

    return k(psrc, pdst, cmat, from_idx, to_idx)


def _deg_stage(to_idx, N, NP, E):
    RPT = NP // _NSUB
    ZR = 128
    EPT = E // _NSUB

    @functools.partial(
        pl.kernel, mesh=_sc_mesh(),
        out_type=jax.ShapeDtypeStruct((NP, _LANES), jnp.float32),
        scratch_types=[
            pltpu.VMEM((_EB,), jnp.int32),
            pltpu.VMEM((_EB, _LANES), jnp.float32),
            pltpu.VMEM((ZR, _LANES), jnp.float32),
            pltpu.VMEM_SHARED((NP, _LANES), jnp.float32),
        ],
    )
    def k(t_hbm, d_hbm, tidx, obuf, zbuf, d_sp):
        c = lax.axis_index("c")
        s = lax.axis_index("s")

        @pl.when(c == 0)
        def _():
            @pl.loop(0, ZR)
            def _(r):
                zbuf[r, pl.ds(0, _LANES)] = jnp.zeros((_LANES,), jnp.float32)

            @pl.loop(0, _EB)
            def _(e):
                obuf[e, pl.ds(0, _LANES)] = jnp.ones((_LANES,), jnp.float32)

            @pl.loop(0, RPT, step=ZR)
            def _(r0):
                pltpu.sync_copy(zbuf, d_sp.at[pl.ds(s * RPT + r0, ZR)])

            plsc.subcore_barrier()

            @pl.loop(0, EPT, step=_EB)
            def _(e0):
                base = s * EPT + e0
                pltpu.sync_copy(t_hbm.at[pl.ds(base, _EB)], tidx)
                pltpu.sync_copy(obuf, d_sp.at[tidx], add=True)

            plsc.subcore_barrier()

            @pl.loop(0, RPT, step=ZR)
            def _(r0):
                r = s * RPT + r0
                pltpu.sync_copy(d_sp.at[pl.ds(r, ZR)], d_hbm.at[pl.ds(r, ZR)])

    return k(to_idx)


def _update(scat, h, deg16, W_msg2, b_msg2, Wu_h, Wu_a, b_upd1, W_upd2,
            b_upd2):
    N, D = h.shape

    def body(s_ref, h_ref, d_ref, wm_ref, bm_ref, wh_ref, wa_ref, b1_ref,
             w2_ref, b2_ref, o_ref):
        agg = (jnp.dot(s_ref[...], wm_ref[...],
                       preferred_element_type=jnp.float32)
               + d_ref[:, 0:1] * bm_ref[...])
        t = jnp.maximum(
            jnp.dot(h_ref[...], wh_ref[...],
                    preferred_element_type=jnp.float32)
            + jnp.dot(agg, wa_ref[...], preferred_element_type=jnp.float32)
            + b1_ref[...], 0.0)
        o_ref[...] = jnp.dot(t, w2_ref[...],
                             preferred_element_type=jnp.float32) + b2_ref[...]

    full = lambda a, b: pl.BlockSpec((a, b), lambda r: (0, 0))
    return _tc_call(
        body, (N // _RB,),
        [pl.BlockSpec((_RB, D), lambda r: (r, 0)),
         pl.BlockSpec((_RB, D), lambda r: (r, 0)),
         pl.BlockSpec((_RB, _LANES), lambda r: (r, 0)),
         full(D, D), full(1, D), full(D, D), full(D, D), full(1, D),
         full(D, D), full(1, D)],
        jax.ShapeDtypeStruct((N, D), jnp.float32),
        pl.BlockSpec((_RB, D), lambda r: (r, 0)),
    )(scat, h, deg16, W_msg2, b_msg2.reshape(1, D), Wu_h, Wu_a,
      b_upd1.reshape(1, D), W_upd2, b_upd2.reshape(1, D))


def _gated_agg(h, gidx3, W_gate, b_gate, W_trans, b_trans):
    N, D = h.shape

    def body(h_ref, g_ref, wg_ref, bg_ref, wt_ref, bt_ref, o_ref):
        hb = h_ref[...]
        gates = jax.nn.sigmoid(
            jnp.dot(hb, wg_ref[...], preferred_element_type=jnp.float32)
            + bg_ref[...])
        trans = jnp.dot(hb, wt_ref[...],
                        preferred_element_type=jnp.float32) + bt_ref[...]
        x = gates * trans
        gid = g_ref[0]
        seg = lax.broadcasted_iota(jnp.int32, (_G, _RB), 0)
        oh = jnp.where(gid == seg, 1.0, 0.0)
        p = jnp.dot(oh, x, preferred_element_type=jnp.float32)

        @pl.when(pl.program_id(0) == 0)
        def _():
            o_ref[...] = jnp.zeros((_G, D), jnp.float32)

        o_ref[...] += p

    full = lambda a, b: pl.BlockSpec((a, b), lambda r: (0, 0))
    return _tc_call(
        body, (N // _RB,),
        [pl.BlockSpec((_RB, D), lambda r: (r, 0)),
         pl.BlockSpec((1, 1, _RB), lambda r: (r, 0, 0)),
         full(D, D), full(1, D), full(D, D), full(1, D)],
        jax.ShapeDtypeStruct((_G, D), jnp.float32),
        pl.BlockSpec((_G, D), lambda r: (0, 0)),
    )(h, gidx3, W_gate, b_gate.reshape(1, D), W_trans, b_trans.reshape(1, D))


def _out_mlp(g, W_out1, b_out1, W_out2, b_out2):
    D = W_out1.shape[0]

    def body(g_ref, w1_ref, b1_ref, w2_ref, b2_ref, o_ref):
        t = jnp.maximum(
            jnp.dot(g_ref[...], w1_ref[...],
                    preferred_element_type=jnp.float32) + b1_ref[...], 0.0)
        o_ref[...] = jnp.dot(t, w2_ref[...],
                             preferred_element_type=jnp.float32) + b2_ref[...]

    full = lambda a, b: pl.BlockSpec((a, b), lambda r: (0, 0))
    return _tc_call(
        body, (1,),
        [full(_G, D), full(D, D), full(1, D), full(D, D), full(1, D)],
        jax.ShapeDtypeStruct((_G, D), jnp.float32),
        full(_G, D),
    )(g, W_out1, b_out1.reshape(1, D), W_out2, b_out2.reshape(1, D))


def kernel(node, edge, edge_index, graph_idx, W_n1, b_n1, W_n2, b_n2, W_e1,
           b_e1, W_e2, b_e2, W_msg1, b_msg1, W_msg2, b_msg2, W_upd1, b_upd1,
           W_upd2, b_upd2, W_gate, b_gate, W_trans, b_trans, W_out1, b_out1,
           W_out2, b_out2):
    N = node.shape[0]
    E = edge.shape[0]
    D = W_n2.shape[1]
    L = W_msg1.shape[0]
    HALF = D // 2
    RPT = (((N + _NSUB - 1) // _NSUB + 127) // 128) * 128
    NP = RPT * _NSUB

    from_idx = edge_index[0].astype(jnp.int32)
    to_idx = edge_index[1].astype(jnp.int32)

    h = _enc_nodes(node, W_n1, b_n1, W_n2, b_n2)
    A = _enc_edge_pre(edge, W_e1, b_e1)
    M_all, c0_all = _fold_edge_weights(W_e2, b_e2, W_msg1[:, 2 * D:, :],
                                       b_msg1)
    deg16 = _deg_stage(to_idx, N, NP, E)[:N]

    Cs = [_cmat(A, M_all[i], c0_all[i]) for i in range(L)]

    for i in range(L):
        W_src = W_msg1[i, :D, :]
        W_dst = W_msg1[i, D:2 * D, :]
        C = Cs[i]
        psrc, pdst = _pmat(h, W_src, W_dst)
        S = _edge_stage(psrc, pdst, C, from_idx, to_idx, N, NP, E, HALF)
        scat = jnp.swapaxes(S.reshape(2, NP, HALF)[:, :N, :], 0,
                            1).reshape(N, D)
        h = _update(scat, h, deg16, W_msg2[i], b_msg2[i],
                    W_upd1[i, :D, :], W_upd1[i, D:, :], b_upd1[i],
                    W_upd2[i], b_upd2[i])

    gidx3 = graph_idx.astype(jnp.int32).reshape(N // _RB, 1, _RB)
    g = _gated_agg(h, gidx3, W_gate, b_gate, W_trans, b_trans)
    return _out_mlp(g, W_out1, b_out1, W_out2, b_out2)


# speedup vs baseline: 1.0131x; 1.0131x over previous
import functools

import jax
import jax.numpy as jnp
from jax import lax
from jax.experimental import pallas as pl
from jax.experimental.pallas import tpu as pltpu
from jax.experimental.pallas import tpu_sc as plsc

_G = 64
_RB = 2000
_EB = 80
_NSUB = 16
_NCORE = 2
_LANES = 16


def _tc_call(body, grid, in_specs, out_shape, out_specs):
    return pl.pallas_call(
        body, grid=grid, in_specs=in_specs, out_shape=out_shape,
        out_specs=out_specs)


def _enc_nodes(node, W_n1, b_n1, W_n2, b_n2):
    N = node.shape[0]
    H = W_n1.shape[1]
    D = W_n2.shape[1]

    def body(n_ref, w1_ref, b1_ref, w2_ref, b2_ref, o_ref):
        t = jnp.maximum(n_ref[...] * w1_ref[...] + b1_ref[...], 0.0)
        o_ref[...] = jnp.dot(t, w2_ref[...],
                             preferred_element_type=jnp.float32) + b2_ref[...]

    return _tc_call(
        body, (N // _RB,),
        [pl.BlockSpec((_RB, 1), lambda r: (r, 0)),
         pl.BlockSpec((1, H), lambda r: (0, 0)),
         pl.BlockSpec((1, H), lambda r: (0, 0)),
         pl.BlockSpec((H, D), lambda r: (0, 0)),
         pl.BlockSpec((1, D), lambda r: (0, 0))],
        jax.ShapeDtypeStruct((N, D), jnp.float32),
        pl.BlockSpec((_RB, D), lambda r: (r, 0)),
    )(node, W_n1, b_n1.reshape(1, H), W_n2, b_n2.reshape(1, D))


def _enc_edge_pre(edge, W_e1, b_e1):
    E = edge.shape[0]
    H = W_e1.shape[1]

    def body(e_ref, w1_ref, b1_ref, o_ref):
        o_ref[...] = jnp.maximum(e_ref[...] * w1_ref[...] + b1_ref[...], 0.0)

    return _tc_call(
        body, (E // _RB,),
        [pl.BlockSpec((_RB, 1), lambda r: (r, 0)),
         pl.BlockSpec((1, H), lambda r: (0, 0)),
         pl.BlockSpec((1, H), lambda r: (0, 0))],
        jax.ShapeDtypeStruct((E, H), jnp.float32),
        pl.BlockSpec((_RB, H), lambda r: (r, 0)),
    )(edge, W_e1, b_e1.reshape(1, H))


def _fold_edge_weights(W_e2, b_e2, W_ef, b_msg1):
    L, D, _ = W_ef.shape
    H = W_e2.shape[0]

    def body(we2_ref, wef_ref, be2_ref, bm1_ref, m_ref, c0_ref):
        wef = wef_ref[0]
        m_ref[...] = jnp.dot(we2_ref[...], wef,
                             preferred_element_type=jnp.float32)[None]
        c0_ref[...] = (jnp.dot(be2_ref[...], wef,
                               preferred_element_type=jnp.float32)
                       + bm1_ref[0])[None]

    return pl.pallas_call(
        body, grid=(L,),
        in_specs=[pl.BlockSpec((H, D), lambda i: (0, 0)),
                  pl.BlockSpec((1, D, D), lambda i: (i, 0, 0)),
                  pl.BlockSpec((1, D), lambda i: (0, 0)),
                  pl.BlockSpec((1, 1, D), lambda i: (i, 0, 0))],
        out_shape=[jax.ShapeDtypeStruct((L, H, D), jnp.float32),
                   jax.ShapeDtypeStruct((L, 1, D), jnp.float32)],
        out_specs=[pl.BlockSpec((1, H, D), lambda i: (i, 0, 0)),
                   pl.BlockSpec((1, 1, D), lambda i: (i, 0, 0))],
    )(W_e2, W_ef, b_e2.reshape(1, D), b_msg1.reshape(L, 1, D))


def _cmat(A, M_i, c0_i):
    E, H = A.shape
    D = M_i.shape[1]
    HALF = D // 2

    def body(a_ref, m_ref, c0_ref, o_ref):
        o_ref[...] = jnp.dot(a_ref[...], m_ref[...],
                             preferred_element_type=jnp.float32) + c0_ref[...]

    return _tc_call(
        body, (E // _RB, 2),
        [pl.BlockSpec((_RB, H), lambda r, c: (r, 0)),
         pl.BlockSpec((H, HALF), lambda r, c: (0, c)),
         pl.BlockSpec((1, HALF), lambda r, c: (0, c))],
        jax.ShapeDtypeStruct((2 * E, HALF), jnp.float32),
        pl.BlockSpec((_RB, HALF), lambda r, c: (c * (E // _RB) + r, 0)),
    )(A, M_i, c0_i)


def _pmat(h, W_src, W_dst):
    N, D = h.shape
    HALF = D // 2

    def body(h_ref, ws_ref, wd_ref, os_ref, od_ref):
        hb = h_ref[...]
        os_ref[...] = jnp.dot(hb, ws_ref[...],
                              preferred_element_type=jnp.float32)
        od_ref[...] = jnp.dot(hb, wd_ref[...],
                              preferred_element_type=jnp.float32)

    nb = N // _RB
    return pl.pallas_call(
        body, grid=(nb, 2),
        in_specs=[pl.BlockSpec((_RB, D), lambda r, c: (r, 0)),
                  pl.BlockSpec((D, HALF), lambda r, c: (0, c)),
                  pl.BlockSpec((D, HALF), lambda r, c: (0, c))],
        out_shape=[jax.ShapeDtypeStruct((2 * N, HALF), jnp.float32),
                   jax.ShapeDtypeStruct((2 * N, HALF), jnp.float32)],
        out_specs=[pl.BlockSpec((_RB, HALF), lambda r, c: (c * nb + r, 0)),
                   pl.BlockSpec((_RB, HALF), lambda r, c: (c * nb + r, 0))],
    )(h, W_src, W_dst)


def _sc_mesh():
    return plsc.VectorSubcoreMesh(core_axis_name="c", subcore_axis_name="s")


def _edge_stage(psrc, pdst, cmat, from_idx, to_idx, N, NP, E, HALF):
    RPT = NP // _NSUB
    ZR = 128
    EPT = E // _NSUB

    NB = EPT // _EB
    QB = 16
    NQ = _EB // QB

    @functools.partial(
        pl.kernel, mesh=_sc_mesh(),
        out_type=jax.ShapeDtypeStruct((2 * NP, HALF), jnp.float32),
        scratch_types=[
            pltpu.VMEM((_EB,), jnp.int32),
            pltpu.VMEM((_EB,), jnp.int32),
            pltpu.VMEM((_EB,), jnp.int32),
            pltpu.VMEM((_EB, HALF), jnp.float32),
            pltpu.VMEM((_EB, HALF), jnp.float32),
            pltpu.VMEM((_EB,), jnp.int32),
            pltpu.VMEM((_EB,), jnp.int32),
            pltpu.VMEM((_EB,), jnp.int32),
            pltpu.VMEM((_EB, HALF), jnp.float32),
            pltpu.VMEM((_EB, HALF), jnp.float32),
            pltpu.VMEM((_EB,), jnp.int32),
            pltpu.VMEM((_EB,), jnp.int32),
            pltpu.VMEM((16, HALF), jnp.float32),
            pltpu.VMEM((16, HALF), jnp.float32),
            pltpu.SemaphoreType.DMA,
            pltpu.SemaphoreType.DMA,
            pltpu.SemaphoreType.DMA,
            pltpu.SemaphoreType.DMA,
            pltpu.SemaphoreType.DMA,
            pltpu.SemaphoreType.DMA,
            pltpu.SemaphoreType.DMA,
            pltpu.SemaphoreType.DMA,
            pltpu.SemaphoreType.DMA,
            pltpu.VMEM_SHARED((NP, HALF), jnp.float32),
        ],
    )
    def k(psrc_hbm, pdst_hbm, c_hbm, f_hbm, t_hbm, s_hbm,
          fidxA, tidxA, didxA, sbufA, dbufA,
          fidxB, tidxB, didxB, sbufB, dbufB,
          stidxA, stidxB, cbuf0, cbuf1,
          slA, sgsA, sgdA, slB, sgsB, sgdB, scc, sscA, sscB, s_sp):
        c = lax.axis_index("c")
        s = lax.axis_index("s")
        noff = c * N
        poff = c * NP
        eoff = c * E
        ebase = s * EPT

        setA = (fidxA, tidxA, didxA, sbufA, dbufA, slA, sgsA, sgdA,
                stidxA, sscA)
        setB = (fidxB, tidxB, didxB, sbufB, dbufB, slB, sgsB, sgdB,
                stidxB, sscB)

        def scat_wait(st):
            pltpu.make_async_copy(st[3], s_sp.at[st[8]], st[9]).wait()

        def lin_copies(st, blk):
            fidx, tidx = st[0], st[1]
            sl = st[5]
            base = ebase + blk * _EB
            return (pltpu.make_async_copy(f_hbm.at[pl.ds(base, _EB)], fidx,
                                          sl),
                    pltpu.make_async_copy(t_hbm.at[pl.ds(base, _EB)], tidx,
                                          sl))

        def lin_start(st, blk):
            for cp in lin_copies(st, blk):
                cp.start()

        def lin_wait(st, blk):
            for cp in lin_copies(st, blk):
                cp.wait()

        def gather_copies(st):
            fidx, didx, sbuf, dbuf, sgs, sgd = (st[0], st[2], st[3], st[4],
                                                st[6], st[7])
            return (pltpu.make_async_copy(psrc_hbm.at[fidx], sbuf, sgs),
                    pltpu.make_async_copy(pdst_hbm.at[didx], dbuf, sgd))

        def adjust_and_issue(st, wait_scat):
            if isinstance(wait_scat, bool):
                if wait_scat:
                    scat_wait(st)
            else:
                @pl.when(wait_scat)
                def _():
                    scat_wait(st)

            fidx, tidx, didx = st[0], st[1], st[2]
            for i in range(0, _EB, _LANES):
                sl = pl.ds(i, _LANES)
                fidx[sl] = fidx[sl] + noff
                didx[sl] = tidx[sl] + noff
            for cp in gather_copies(st):
                cp.start()

        def c_copy(blk, q, buf):
            return pltpu.make_async_copy(
                c_hbm.at[pl.ds(eoff + ebase + blk * _EB + q * QB, QB)],
                buf, scc)

        def compute_scatter(st, blk, nxt_blk, prefetch_next, phase):
            tidx, sbuf, dbuf, stidx, ssc = st[1], st[3], st[4], st[8], st[9]
            bufs = (cbuf0, cbuf1)
            for cp in gather_copies(st):
                cp.wait()
            for i in range(0, _EB, _LANES):
                sl = pl.ds(i, _LANES)
                stidx[sl] = tidx[sl]
            for q in range(NQ):
                cb = bufs[(phase + q) % 2]
                nb = bufs[(phase + q + 1) % 2]
                c_copy(blk, q, cb).wait()
                if q < NQ - 1:
                    c_copy(blk, q + 1, nb).start()
                elif isinstance(prefetch_next, bool):
                    if prefetch_next:
                        c_copy(nxt_blk, 0, nb).start()
                else:
                    @pl.when(prefetch_next)
                    def _():
                        c_copy(nxt_blk, 0, nb).start()

                @pl.loop(0, QB, step=2)
                def _(e0):
                    for u in range(2):
                        e = e0 + (q * QB + u)
                        ec = e0 + u
                        for j in range(0, HALF, _LANES):
                            sl = pl.ds(j, _LANES)
                            sbuf[e, sl] = jnp.maximum(
                                sbuf[e, sl] + dbuf[e, sl] + cb[ec, sl], 0.0)

            pltpu.async_copy(sbuf, s_sp.at[stidx], ssc, add=True)

        @pl.loop(0, _EB)
        def _(r):
            for j in range(0, HALF, _LANES):
                sbufA[r, pl.ds(j, _LANES)] = jnp.zeros((_LANES,), jnp.float32)

        @pl.loop(0, RPT, step=_EB)
        def _(r0):
            pltpu.sync_copy(sbufA, s_sp.at[pl.ds(s * RPT + r0, _EB)])

        plsc.subcore_barrier()

        lin_start(setA, 0)
        c_copy(0, 0, cbuf0).start()
        lin_wait(setA, 0)
        adjust_and_issue(setA, False)
        compute_scatter(setA, 0, 1, True, 0)
        lin_start(setA, 1)
        lin_wait(setA, 1)
        adjust_and_issue(setA, True)
        lin_start(setB, 2)

        @pl.loop(0, NB - 1, step=2)
        def _(j):
            lin_wait(setB, 2 + j)
            adjust_and_issue(setB, j > 0)
            compute_scatter(setA, 1 + j, 2 + j, True, 1)

            @pl.when(j < NB - 3)
            def _():
                lin_start(setA, 3 + j)
                lin_wait(setA, 3 + j)
                adjust_and_issue(setA, True)

            compute_scatter(setB, 2 + j, 3 + j, j < NB - 3, 0)

            @pl.when(j < NB - 4)
            def _():
                lin_start(setB, 4 + j)

        scat_wait(setA)
        scat_wait(setB)
        plsc.subcore_barrier()

        @pl.loop(0, RPT, step=ZR)
        def _(r0):
            r = s * RPT + r0
            pltpu.sync_copy(s_sp.at[pl.ds(r, ZR)],
                            s_hbm.at[pl.ds(poff + r, ZR)])

    return k(psrc, pdst, cmat, from_idx, to_idx)


def _deg_stage(to_idx, N, NP, E):
    RPT = NP // _NSUB
    ZR = 128
    EPT = E // _NSUB

    @functools.partial(
        pl.kernel, mesh=_sc_mesh(),
        out_type=jax.ShapeDtypeStruct((NP, _LANES), jnp.float32),
        scratch_types=[
            pltpu.VMEM((_EB,), jnp.int32),
            pltpu.VMEM((_EB, _LANES), jnp.float32),
            pltpu.VMEM((ZR, _LANES), jnp.float32),
            pltpu.VMEM_SHARED((NP, _LANES), jnp.float32),
        ],
    )
    def k(t_hbm, d_hbm, tidx, obuf, zbuf, d_sp):
        c = lax.axis_index("c")
        s = lax.axis_index("s")

        @pl.when(c == 0)
        def _():
            @pl.loop(0, ZR)
            def _(r):
                zbuf[r, pl.ds(0, _LANES)] = jnp.zeros((_LANES,), jnp.float32)

            @pl.loop(0, _EB)
            def _(e):
                obuf[e, pl.ds(0, _LANES)] = jnp.ones((_LANES,), jnp.float32)

            @pl.loop(0, RPT, step=ZR)
            def _(r0):
                pltpu.sync_copy(zbuf, d_sp.at[pl.ds(s * RPT + r0, ZR)])

            plsc.subcore_barrier()

            @pl.loop(0, EPT, step=_EB)
            def _(e0):
                base = s * EPT + e0
                pltpu.sync_copy(t_hbm.at[pl.ds(base, _EB)], tidx)
                pltpu.sync_copy(obuf, d_sp.at[tidx], add=True)

            plsc.subcore_barrier()

            @pl.loop(0, RPT, step=ZR)
            def _(r0):
                r = s * RPT + r0
                pltpu.sync_copy(d_sp.at[pl.ds(r, ZR)], d_hbm.at[pl.ds(r, ZR)])

    return k(to_idx)


def _update(scat, h, deg16, W_msg2, b_msg2, Wu_h, Wu_a, b_upd1, W_upd2,
            b_upd2):
    N, D = h.shape

    def body(s_ref, h_ref, d_ref, wm_ref, bm_ref, wh_ref, wa_ref, b1_ref,
             w2_ref, b2_ref, o_ref):
        agg = (jnp.dot(s_ref[...], wm_ref[...],
                       preferred_element_type=jnp.float32)
               + d_ref[:, 0:1] * bm_ref[...])
        t = jnp.maximum(
            jnp.dot(h_ref[...], wh_ref[...],
                    preferred_element_type=jnp.float32)
            + jnp.dot(agg, wa_ref[...], preferred_element_type=jnp.float32)
            + b1_ref[...], 0.0)
        o_ref[...] = jnp.dot(t, w2_ref[...],
                             preferred_element_type=jnp.float32) + b2_ref[...]

    full = lambda a, b: pl.BlockSpec((a, b), lambda r: (0, 0))
    return _tc_call(
        body, (N // _RB,),
        [pl.BlockSpec((_RB, D), lambda r: (r, 0)),
         pl.BlockSpec((_RB, D), lambda r: (r, 0)),
         pl.BlockSpec((_RB, _LANES), lambda r: (r, 0)),
         full(D, D), full(1, D), full(D, D), full(D, D), full(1, D),
         full(D, D), full(1, D)],
        jax.ShapeDtypeStruct((N, D), jnp.float32),
        pl.BlockSpec((_RB, D), lambda r: (r, 0)),
    )(scat, h, deg16, W_msg2, b_msg2.reshape(1, D), Wu_h, Wu_a,
      b_upd1.reshape(1, D), W_upd2, b_upd2.reshape(1, D))


def _gated_agg(h, gidx3, W_gate, b_gate, W_trans, b_trans):
    N, D = h.shape

    def body(h_ref, g_ref, wg_ref, bg_ref, wt_ref, bt_ref, o_ref):
        hb = h_ref[...]
        gates = jax.nn.sigmoid(
            jnp.dot(hb, wg_ref[...], preferred_element_type=jnp.float32)
            + bg_ref[...])
        trans = jnp.dot(hb, wt_ref[...],
                        preferred_element_type=jnp.float32) + bt_ref[...]
        x = gates * trans
        gid = g_ref[0]
        seg = lax.broadcasted_iota(jnp.int32, (_G, _RB), 0)
        oh = jnp.where(gid == seg, 1.0, 0.0)
        p = jnp.dot(oh, x, preferred_element_type=jnp.float32)

        @pl.when(pl.program_id(0) == 0)
        def _():
            o_ref[...] = jnp.zeros((_G, D), jnp.float32)

        o_ref[...] += p

    full = lambda a, b: pl.BlockSpec((a, b), lambda r: (0, 0))
    return _tc_call(
        body, (N // _RB,),
        [pl.BlockSpec((_RB, D), lambda r: (r, 0)),
         pl.BlockSpec((1, 1, _RB), lambda r: (r, 0, 0)),
         full(D, D), full(1, D), full(D, D), full(1, D)],
        jax.ShapeDtypeStruct((_G, D), jnp.float32),
        pl.BlockSpec((_G, D), lambda r: (0, 0)),
    )(h, gidx3, W_gate, b_gate.reshape(1, D), W_trans, b_trans.reshape(1, D))


def _out_mlp(g, W_out1, b_out1, W_out2, b_out2):
    D = W_out1.shape[0]

    def body(g_ref, w1_ref, b1_ref, w2_ref, b2_ref, o_ref):
        t = jnp.maximum(
            jnp.dot(g_ref[...], w1_ref[...],
                    preferred_element_type=jnp.float32) + b1_ref[...], 0.0)
        o_ref[...] = jnp.dot(t, w2_ref[...],
                             preferred_element_type=jnp.float32) + b2_ref[...]

    full = lambda a, b: pl.BlockSpec((a, b), lambda r: (0, 0))
    return _tc_call(
        body, (1,),
        [full(_G, D), full(D, D), full(1, D), full(D, D), full(1, D)],
        jax.ShapeDtypeStruct((_G, D), jnp.float32),
        full(_G, D),
    )(g, W_out1, b_out1.reshape(1, D), W_out2, b_out2.reshape(1, D))


def kernel(node, edge, edge_index, graph_idx, W_n1, b_n1, W_n2, b_n2, W_e1,
           b_e1, W_e2, b_e2, W_msg1, b_msg1, W_msg2, b_msg2, W_upd1, b_upd1,
           W_upd2, b_upd2, W_gate, b_gate, W_trans, b_trans, W_out1, b_out1,
           W_out2, b_out2):
    N = node.shape[0]
    E = edge.shape[0]
    D = W_n2.shape[1]
    L = W_msg1.shape[0]
    HALF = D // 2
    RPT = (((N + _NSUB - 1) // _NSUB + 127) // 128) * 128
    NP = RPT * _NSUB

    from_idx = edge_index[0].astype(jnp.int32)
    to_idx = edge_index[1].astype(jnp.int32)

    h = _enc_nodes(node, W_n1, b_n1, W_n2, b_n2)
    A = _enc_edge_pre(edge, W_e1, b_e1)
    M_all, c0_all = _fold_edge_weights(W_e2, b_e2, W_msg1[:, 2 * D:, :],
                                       b_msg1)
    deg16 = _deg_stage(to_idx, N, NP, E)[:N]

    Cs = [_cmat(A, M_all[i], c0_all[i]) for i in range(L)]

    for i in range(L):
        W_src = W_msg1[i, :D, :]
        W_dst = W_msg1[i, D:2 * D, :]
        C = Cs[i]
        psrc, pdst = _pmat(h, W_src, W_dst)
        S = _edge_stage(psrc, pdst, C, from_idx, to_idx, N, NP, E, HALF)
        scat = jnp.swapaxes(S.reshape(2, NP, HALF)[:, :N, :], 0,
                            1).reshape(N, D)
        h = _update(scat, h, deg16, W_msg2[i], b_msg2[i],
                    W_upd1[i, :D, :], W_upd1[i, D:, :], b_upd1[i],
                    W_upd2[i], b_upd2[i])

    gidx3 = graph_idx.astype(jnp.int32).reshape(N // _RB, 1, _RB)
    g = _gated_agg(h, gidx3, W_gate, b_gate, W_trans, b_trans)
    return _out_mlp(g, W_out1, b_out1, W_out2, b_out2)
